# baseline (device time: 19927 ns/iter reference)
import jax
import jax.numpy as jnp
from jax import lax
from jax.experimental import pallas as pl
from jax.experimental.pallas import tpu as pltpu

_CQ = 8
_HF = _CQ // 2


def kernel(x):
    m, n = x.shape
    qrows = m // 4
    rpc = qrows // _CQ

    def body(x_ref, out_ref, zbuf, z_send, z_recv, x_send, x_recv,
             y_send, y_recv):
        mx = lax.axis_index("x")
        my = lax.axis_index("y")
        mz = lax.axis_index("z")
        q = 2 * my + mx
        qx = 2 * my + (1 - mx)
        qy = 2 * (1 - my) + mx
        zp = (mx, my, 1 - mz)
        xp = (1 - mx, my, mz)
        yp = (mx, 1 - my, mz)

        barrier_sem = pltpu.get_barrier_semaphore()
        for nbr in (zp, xp, yp):
            pl.semaphore_signal(
                barrier_sem, inc=1,
                device_id=nbr, device_id_type=pl.DeviceIdType.MESH,
            )
        pl.semaphore_wait(barrier_sem, 3)

        row0 = q * qrows
        rowx = qx * qrows
        rowy = qy * qrows

        def copy(rows, send_sem, recv_sem, dev):
            return pltpu.make_async_remote_copy(
                src_ref=out_ref.at[pl.ds(rows, rpc), :],
                dst_ref=out_ref.at[pl.ds(rows, rpc), :],
                send_sem=send_sem,
                recv_sem=recv_sem,
                device_id=dev,
                device_id_type=pl.DeviceIdType.MESH,
            )

        z_rdmas = []
        for c in range(_CQ):
            r = pltpu.make_async_remote_copy(
                src_ref=x_ref.at[pl.ds(row0 + c * rpc, rpc), :],
                dst_ref=zbuf.at[pl.ds(c * rpc, rpc), :],
                send_sem=z_send.at[c],
                recv_sem=z_recv.at[c],
                device_id=zp,
                device_id_type=pl.DeviceIdType.MESH,
            )
            r.start()
            z_rdmas.append(r)

        xq = []
        yq = []
        for c in range(_CQ):
            z_rdmas[c].wait_recv()
            out_ref[pl.ds(row0 + c * rpc, rpc), :] = (
                x_ref[pl.ds(row0 + c * rpc, rpc), :]
                + zbuf[pl.ds(c * rpc, rpc), :]
            )
            rx = copy(row0 + c * rpc, x_send.at[c], x_recv.at[c], xp)
            rx.start()
            xq.append(rx)
            ry = copy(row0 + c * rpc, y_send.at[c], y_recv.at[c], yp)
            ry.start()
            yq.append(ry)

        xf = []
        yf = []
        for c in range(_HF):
            yq[c].wait_recv()
            rf = copy(rowy + c * rpc, x_send.at[_CQ + c],
                      x_recv.at[_CQ + c], xp)
            rf.start()
            xf.append(rf)
            xq[_HF + c].wait_recv()
            rf = copy(rowx + (_HF + c) * rpc, y_send.at[_CQ + c],
                      y_recv.at[_CQ + c], yp)
            rf.start()
            yf.append(rf)

        for c in range(_HF):
            xq[c].wait_recv()
            yq[_HF + c].wait_recv()
        for r in xf + yf:
            r.wait_recv()
        for r in z_rdmas + xq + yq + xf + yf:
            r.wait_send()

    return pl.pallas_call(
        body,
        out_shape=jax.ShapeDtypeStruct((m, n), x.dtype),
        in_specs=[pl.BlockSpec(memory_space=pltpu.VMEM)],
        out_specs=pl.BlockSpec(memory_space=pltpu.VMEM),
        scratch_shapes=[
            pltpu.VMEM((qrows, n), x.dtype),
            pltpu.SemaphoreType.DMA((_CQ,)),
            pltpu.SemaphoreType.DMA((_CQ,)),
            pltpu.SemaphoreType.DMA((_CQ + _HF,)),
            pltpu.SemaphoreType.DMA((_CQ + _HF,)),
            pltpu.SemaphoreType.DMA((_CQ + _HF,)),
            pltpu.SemaphoreType.DMA((_CQ + _HF,)),
        ],
        compiler_params=pltpu.CompilerParams(collective_id=0),
    )(x)
